# baseline (speedup 1.0000x reference)
import jax
import jax.numpy as jnp
from jax import lax
from jax.experimental import pallas as pl
from jax.experimental.pallas import tpu as pltpu
from jax.experimental.pallas import tpu_sc as plsc

NUM_USERS = 100000
NUM_ITEMS = 100000
EMB = 128
BATCH = 16384
LANES = 16
GROUPS = EMB // LANES

_info = plsc.get_sparse_core_info()
NC, NS = _info.num_cores, _info.num_subcores
NW = NC * NS
B_PER_W = BATCH // NW
CHUNK = 128
NCHUNK = B_PER_W // CHUNK
NACC = 4


def _bpr_body(user_hbm, pos_hbm, neg_hbm, uemb_hbm, iemb_hbm,
              pos_out_hbm, neg_out_hbm,
              uidx_v, pidx_v, nidx_v, u_rows, p_rows, n_rows,
              t_pos, t_neg, pos_o, neg_o, sem0, sem1):
    wid = lax.axis_index("s") * NC + lax.axis_index("c")
    base = wid * B_PER_W
    sems = (sem0, sem1)

    ci = pltpu.async_copy(user_hbm.at[pl.ds(base, B_PER_W)], uidx_v, sem0)
    cp = pltpu.async_copy(pos_hbm.at[pl.ds(base, B_PER_W)], pidx_v, sem0)
    cn = pltpu.async_copy(neg_hbm.at[pl.ds(base, B_PER_W)], nidx_v, sem0)
    ci.wait()
    cp.wait()
    cn.wait()

    def start(c):
        buf = c % 2
        dst = pl.ds(buf * CHUNK, CHUNK)
        src = pl.ds(c * CHUNK, CHUNK)
        sem = sems[buf]
        return (
            pltpu.async_copy(uemb_hbm.at[uidx_v.at[src]], u_rows.at[dst], sem),
            pltpu.async_copy(iemb_hbm.at[pidx_v.at[src]], p_rows.at[dst], sem),
            pltpu.async_copy(iemb_hbm.at[nidx_v.at[src]], n_rows.at[dst], sem),
        )

    lane = lax.iota(jnp.int32, LANES)
    zero = jnp.zeros((LANES,), jnp.float32)

    def compute(c):
        buf = c % 2

        def grp_body(grp, _):
            rbase = buf * CHUNK + grp * LANES
            for j in range(LANES):
                r = rbase + j
                pa = [None, None]
                na = [None, None]
                for g in range(GROUPS):
                    u = u_rows[r, pl.ds(g * LANES, LANES)]
                    p = p_rows[r, pl.ds(g * LANES, LANES)]
                    n = n_rows[r, pl.ds(g * LANES, LANES)]
                    a = g % 2
                    pa[a] = u * p if pa[a] is None else pa[a] + u * p
                    na[a] = u * n if na[a] is None else na[a] + u * n
                t_pos[j, pl.ds(0, LANES)] = pa[0] + pa[1]
                t_neg[j, pl.ds(0, LANES)] = na[0] + na[1]

            pv = [zero] * NACC
            nv = [zero] * NACC
            for k in range(LANES):
                col = jnp.broadcast_to(jnp.int32(k), (LANES,))
                tp = plsc.load_gather(t_pos, [lane, col])
                tn = plsc.load_gather(t_neg, [lane, col])
                pv[k % NACC] = pv[k % NACC] + tp
                nv[k % NACC] = nv[k % NACC] + tn
            pos_vec = (pv[0] + pv[1]) + (pv[2] + pv[3])
            neg_vec = (nv[0] + nv[1]) + (nv[2] + nv[3])
            pos_o[pl.ds(c * CHUNK + grp * LANES, LANES)] = pos_vec
            neg_o[pl.ds(c * CHUNK + grp * LANES, LANES)] = neg_vec
            return 0

        lax.fori_loop(0, CHUNK // LANES, grp_body, 0)

    inflight = start(0)
    for c in range(NCHUNK):
        nxt = start(c + 1) if c + 1 < NCHUNK else None
        for d in inflight:
            d.wait()
        compute(c)
        inflight = nxt

    pltpu.sync_copy(pos_o, pos_out_hbm.at[pl.ds(base, B_PER_W)])
    pltpu.sync_copy(neg_o, neg_out_hbm.at[pl.ds(base, B_PER_W)])


@jax.jit
def _bpr(user, pos_item, neg_item, user_emb, item_emb):
    mesh = plsc.VectorSubcoreMesh(core_axis_name="c", subcore_axis_name="s")
    f = pl.kernel(
        _bpr_body,
        out_type=(
            jax.ShapeDtypeStruct((BATCH,), jnp.float32),
            jax.ShapeDtypeStruct((BATCH,), jnp.float32),
        ),
        mesh=mesh,
        compiler_params=pltpu.CompilerParams(needs_layout_passes=False),
        scratch_types=[
            pltpu.VMEM((B_PER_W,), jnp.int32),
            pltpu.VMEM((B_PER_W,), jnp.int32),
            pltpu.VMEM((B_PER_W,), jnp.int32),
            pltpu.VMEM((2 * CHUNK, EMB), jnp.float32),
            pltpu.VMEM((2 * CHUNK, EMB), jnp.float32),
            pltpu.VMEM((2 * CHUNK, EMB), jnp.float32),
            pltpu.VMEM((LANES, LANES + 1), jnp.float32),
            pltpu.VMEM((LANES, LANES + 1), jnp.float32),
            pltpu.VMEM((B_PER_W,), jnp.float32),
            pltpu.VMEM((B_PER_W,), jnp.float32),
            pltpu.SemaphoreType.DMA,
            pltpu.SemaphoreType.DMA,
        ],
    )
    return f(user, pos_item, neg_item, user_emb, item_emb)


def kernel(user, pos_item, neg_item, user_emb, item_emb):
    return _bpr(user.astype(jnp.int32), pos_item.astype(jnp.int32),
                neg_item.astype(jnp.int32), user_emb, item_emb)


# speedup vs baseline: 1.4044x; 1.4044x over previous
import functools

import jax
import jax.numpy as jnp
from jax import lax
from jax.experimental import pallas as pl
from jax.experimental.pallas import tpu as pltpu
from jax.experimental.pallas import tpu_sc as plsc

NUM_USERS = 100000
NUM_ITEMS = 100000
EMB = 128
BATCH = 16384
LANES = 16
GROUPS = EMB // LANES

_info = plsc.get_sparse_core_info()
NC, NS = _info.num_cores, _info.num_subcores
NW = NC * NS
B_PER_W = BATCH // NW
CHUNK = 128
NCHUNK = B_PER_W // CHUNK
NACC = 4
DUNROLL = 16


def _bpr_body(user_hbm, pos_hbm, neg_hbm, uemb_hbm, iemb_hbm,
              pos_out_hbm, neg_out_hbm,
              uidx_v, pidx_v, nidx_v, u_rows, p_rows, n_rows,
              pos_o, neg_o, sem0, sem1):
    wid = lax.axis_index("s") * NC + lax.axis_index("c")
    base = wid * B_PER_W
    sems = (sem0, sem1)

    ci = pltpu.async_copy(user_hbm.at[pl.ds(base, B_PER_W)], uidx_v, sem0)
    cp = pltpu.async_copy(pos_hbm.at[pl.ds(base, B_PER_W)], pidx_v, sem0)
    cn = pltpu.async_copy(neg_hbm.at[pl.ds(base, B_PER_W)], nidx_v, sem0)
    ci.wait()
    cp.wait()
    cn.wait()

    def start(c):
        buf = c % 2
        dst = pl.ds(buf * CHUNK, CHUNK)
        src = pl.ds(c * CHUNK, CHUNK)
        sem = sems[buf]
        return (
            pltpu.async_copy(uemb_hbm.at[uidx_v.at[src]], u_rows.at[dst], sem),
            pltpu.async_copy(iemb_hbm.at[pidx_v.at[src]], p_rows.at[dst], sem),
            pltpu.async_copy(iemb_hbm.at[nidx_v.at[src]], n_rows.at[dst], sem),
        )

    lane = lax.iota(jnp.int32, LANES)
    zero = jnp.zeros((LANES,), jnp.float32)

    def compute(c):
        buf = c % 2

        def body(grp, _):
            row = buf * CHUNK + grp * LANES + lane

            def d_body(db, carry):
                pacc = list(carry[:NACC])
                nacc = list(carry[NACC:])
                for k in range(DUNROLL):
                    col = (jnp.broadcast_to(db * DUNROLL + k, (LANES,))
                           + lane) & (EMB - 1)
                    u = plsc.load_gather(u_rows, [row, col])
                    p = plsc.load_gather(p_rows, [row, col])
                    n = plsc.load_gather(n_rows, [row, col])
                    pacc[k % NACC] = pacc[k % NACC] + u * p
                    nacc[k % NACC] = nacc[k % NACC] + u * n
                return tuple(pacc) + tuple(nacc)

            accs = lax.fori_loop(0, EMB // DUNROLL, d_body,
                                 (zero,) * (2 * NACC))
            pos_vec = (accs[0] + accs[1]) + (accs[2] + accs[3])
            neg_vec = (accs[4] + accs[5]) + (accs[6] + accs[7])
            pos_o[pl.ds(c * CHUNK + grp * LANES, LANES)] = pos_vec
            neg_o[pl.ds(c * CHUNK + grp * LANES, LANES)] = neg_vec
            return 0

        lax.fori_loop(0, CHUNK // LANES, body, 0)

    inflight = start(0)
    for c in range(NCHUNK):
        nxt = start(c + 1) if c + 1 < NCHUNK else None
        for d in inflight:
            d.wait()
        compute(c)
        inflight = nxt

    pltpu.sync_copy(pos_o, pos_out_hbm.at[pl.ds(base, B_PER_W)])
    pltpu.sync_copy(neg_o, neg_out_hbm.at[pl.ds(base, B_PER_W)])


@jax.jit
def _bpr(user, pos_item, neg_item, user_emb, item_emb):
    mesh = plsc.VectorSubcoreMesh(core_axis_name="c", subcore_axis_name="s")
    f = pl.kernel(
        _bpr_body,
        out_type=(
            jax.ShapeDtypeStruct((BATCH,), jnp.float32),
            jax.ShapeDtypeStruct((BATCH,), jnp.float32),
        ),
        mesh=mesh,
        compiler_params=pltpu.CompilerParams(needs_layout_passes=False),
        scratch_types=[
            pltpu.VMEM((B_PER_W,), jnp.int32),
            pltpu.VMEM((B_PER_W,), jnp.int32),
            pltpu.VMEM((B_PER_W,), jnp.int32),
            pltpu.VMEM((2 * CHUNK, EMB), jnp.float32),
            pltpu.VMEM((2 * CHUNK, EMB), jnp.float32),
            pltpu.VMEM((2 * CHUNK, EMB), jnp.float32),
            pltpu.VMEM((B_PER_W,), jnp.float32),
            pltpu.VMEM((B_PER_W,), jnp.float32),
            pltpu.SemaphoreType.DMA,
            pltpu.SemaphoreType.DMA,
        ],
    )
    return f(user, pos_item, neg_item, user_emb, item_emb)


def kernel(user, pos_item, neg_item, user_emb, item_emb):
    return _bpr(user.astype(jnp.int32), pos_item.astype(jnp.int32),
                neg_item.astype(jnp.int32), user_emb, item_emb)
